# baseline (speedup 1.0000x reference)
import functools

import jax
import jax.numpy as jnp
from jax import lax
from jax.experimental import pallas as pl
from jax.experimental.pallas import tpu as pltpu
from jax.experimental.pallas import tpu_sc as plsc

NUM_FIELDS = 26
VOCAB_P1 = 100001
EMB_DIM = 32
BATCH = 16384
TOTAL = BATCH * NUM_FIELDS

NC = 2
NS = 16
L = 16
NW = NC * NS
ROWS_PER_W = TOTAL // NW
CHUNK = 128
NCHUNK = ROWS_PER_W // CHUNK


def _body(xf_hbm, tf_hbm, out_hbm, idx_v, rows_v, sem):
    wid = lax.axis_index("s") * NC + lax.axis_index("c")
    base = wid * ROWS_PER_W

    def chunk_body(i, carry):
        off = base + i * CHUNK
        pltpu.sync_copy(xf_hbm.at[pl.ds(off, CHUNK)], idx_v)
        for j in range(CHUNK // L):
            pos = off + j * L + lax.iota(jnp.int32, L)
            fld = lax.rem(pos, NUM_FIELDS)
            idx_v[pl.ds(j * L, L)] = idx_v[pl.ds(j * L, L)] + fld * VOCAB_P1
        pltpu.async_copy(tf_hbm.at[idx_v], rows_v, sem).wait()
        pltpu.sync_copy(rows_v, out_hbm.at[pl.ds(off, CHUNK)])
        return carry

    lax.fori_loop(0, NCHUNK, chunk_body, 0)


_mesh = plsc.VectorSubcoreMesh(core_axis_name="c", subcore_axis_name="s")

_gather = functools.partial(
    pl.kernel,
    mesh=_mesh,
    out_type=jax.ShapeDtypeStruct((TOTAL, EMB_DIM), jnp.float32),
    scratch_types=[
        pltpu.VMEM((CHUNK,), jnp.int32),
        pltpu.VMEM((CHUNK, EMB_DIM), jnp.float32),
        pltpu.SemaphoreType.DMA,
    ],
    compiler_params=pltpu.CompilerParams(use_tc_tiling_on_sc=False),
)(_body)


def kernel(x, tables):
    xf = x.reshape(TOTAL)
    tf = tables.reshape(NUM_FIELDS * VOCAB_P1, EMB_DIM)
    out = _gather(xf, tf)
    return out.reshape(BATCH, NUM_FIELDS, EMB_DIM)


# speedup vs baseline: 1.0092x; 1.0092x over previous
import functools

import jax
import jax.numpy as jnp
from jax import lax
from jax.experimental import pallas as pl
from jax.experimental.pallas import tpu as pltpu
from jax.experimental.pallas import tpu_sc as plsc

NUM_FIELDS = 26
VOCAB_P1 = 100001
EMB_DIM = 32
BATCH = 16384
TOTAL = BATCH * NUM_FIELDS

NC = 2
NS = 16
L = 16
NW = NC * NS
ROWS_PER_W = TOTAL // NW
CHUNK = 128
NCHUNK = ROWS_PER_W // CHUNK
G = 6
SD = 2
NBUF = G + SD


def _body(x2d_hbm, tf_hbm, out_hbm, idx2d, rows, gsem, ssem):
    wid = lax.axis_index("s") * NC + lax.axis_index("c")
    wbase = wid * ROWS_PER_W

    pltpu.sync_copy(x2d_hbm.at[pl.ds(wid * NCHUNK, NCHUNK)], idx2d)

    def idx_body(c, carry):
        for j in range(CHUNK // L):
            pos = wbase + c * CHUNK + j * L + lax.iota(jnp.int32, L)
            fld = lax.rem(pos, NUM_FIELDS)
            idx2d[c, pl.ds(j * L, L)] = idx2d[c, pl.ds(j * L, L)] + fld * VOCAB_P1
        return carry

    lax.fori_loop(0, NCHUNK, idx_body, 0)

    def fire_gather(c, b):
        pltpu.async_copy(tf_hbm.at[idx2d.at[c]], rows.at[b], gsem)

    def wait_gather(b):
        pltpu.make_async_copy(tf_hbm.at[idx2d.at[0]], rows.at[b], gsem).wait()

    def wait_store(b):
        pltpu.make_async_copy(out_hbm.at[pl.ds(0, CHUNK)], rows.at[b], ssem).wait()

    for j in range(G):
        fire_gather(j, j)

    def main_body(i, carry):
        b = lax.rem(i, NBUF)
        wait_gather(b)
        pltpu.async_copy(rows.at[b], out_hbm.at[pl.ds(wbase + i * CHUNK, CHUNK)], ssem)

        @pl.when(i >= SD)
        def _():
            wait_store(b)

        @pl.when(i + G < NCHUNK)
        def _():
            fire_gather(i + G, lax.rem(i + G, NBUF))

        return carry

    lax.fori_loop(0, NCHUNK, main_body, 0)
    for j in range(SD):
        wait_store(j)


_mesh = plsc.VectorSubcoreMesh(core_axis_name="c", subcore_axis_name="s")

_gather = functools.partial(
    pl.kernel,
    mesh=_mesh,
    out_type=jax.ShapeDtypeStruct((TOTAL, EMB_DIM), jnp.float32),
    scratch_types=[
        pltpu.VMEM((NCHUNK, CHUNK), jnp.int32),
        pltpu.VMEM((NBUF, CHUNK, EMB_DIM), jnp.float32),
        pltpu.SemaphoreType.DMA,
        pltpu.SemaphoreType.DMA,
    ],
    compiler_params=pltpu.CompilerParams(use_tc_tiling_on_sc=False),
)(_body)


def kernel(x, tables):
    x2d = x.reshape(TOTAL // CHUNK, CHUNK)
    tf = tables.reshape(NUM_FIELDS * VOCAB_P1, EMB_DIM)
    out = _gather(x2d, tf)
    return out.reshape(BATCH, NUM_FIELDS, EMB_DIM)


# speedup vs baseline: 11.6725x; 11.5662x over previous
import functools

import jax
import jax.numpy as jnp
from jax import lax
from jax.experimental import pallas as pl
from jax.experimental.pallas import tpu as pltpu
from jax.experimental.pallas import tpu_sc as plsc

NUM_FIELDS = 26
VOCAB_P1 = 100001
EMB_DIM = 32
BATCH = 16384

NC = 2
NS = 16
NW = NC * NS
B_PER_W = BATCH // NW
BB = 8
NCHUNKW = B_PER_W // BB


def _body(x_hbm, tf_hbm, out_hbm, xv, rows, semx, semg, semo):
    wid = lax.axis_index("s") * NC + lax.axis_index("c")
    b0w = wid * B_PER_W

    pltpu.async_copy(x_hbm.at[pl.ds(b0w, BB)], xv.at[0], semx)

    def chunk_body(c, carry):
        buf = lax.rem(c, 2)
        pltpu.make_async_copy(x_hbm.at[pl.ds(0, BB)], xv.at[buf], semx).wait()

        @pl.when(c + 1 < NCHUNKW)
        def _():
            pltpu.async_copy(
                x_hbm.at[pl.ds(b0w + (c + 1) * BB, BB)],
                xv.at[lax.rem(c + 1, 2)], semx)

        @pl.when(c >= 2)
        def _():
            pltpu.make_async_copy(
                out_hbm.at[pl.ds(0, BB)], rows.at[0], semo).wait()

        def row_body(b, carry2):
            v0 = xv[buf, b, pl.ds(0, 16)]
            v1 = xv[buf, b, pl.ds(NUM_FIELDS - 16, 16)]
            for f in range(NUM_FIELDS):
                r = v0[f] if f < 16 else v1[f - (NUM_FIELDS - 16)]
                pltpu.async_copy(tf_hbm.at[f, r], rows.at[buf, b, f], semg)
            return carry2

        lax.fori_loop(0, BB, row_body, 0)
        pltpu.make_async_copy(out_hbm.at[pl.ds(0, BB)], rows.at[buf], semg).wait()
        pltpu.async_copy(rows.at[buf], out_hbm.at[pl.ds(b0w + c * BB, BB)], semo)
        return carry

    lax.fori_loop(0, NCHUNKW, chunk_body, 0)
    for _ in range(2):
        pltpu.make_async_copy(out_hbm.at[pl.ds(0, BB)], rows.at[0], semo).wait()


_mesh = plsc.VectorSubcoreMesh(core_axis_name="c", subcore_axis_name="s")

_gather = functools.partial(
    pl.kernel,
    mesh=_mesh,
    out_type=jax.ShapeDtypeStruct((BATCH, NUM_FIELDS, EMB_DIM), jnp.float32),
    scratch_types=[
        pltpu.VMEM((2, BB, NUM_FIELDS), jnp.int32),
        pltpu.VMEM((2, BB, NUM_FIELDS, EMB_DIM), jnp.float32),
        pltpu.SemaphoreType.DMA,
        pltpu.SemaphoreType.DMA,
        pltpu.SemaphoreType.DMA,
    ],
    compiler_params=pltpu.CompilerParams(use_tc_tiling_on_sc=True),
)(_body)


def kernel(x, tables):
    return _gather(x, tables)
